# baseline (speedup 1.0000x reference)
import functools

import jax
import jax.numpy as jnp
from jax.experimental import pallas as pl
from jax.experimental.pallas import tpu as pltpu

CHANNEL = 64
SEG = 4
K = 128
LEVELS = 3
D = CHANNEL // SEG
KT = SEG * K


def _mm(a, b, dims):
    return jax.lax.dot_general(
        a, b, (dims, ((), ())), preferred_element_type=jnp.float32)


def _body(x_ref, w_enc, b_enc_r, w_q, b_q_r, w_lat, b_lat_r, w_deq, b_deq_r,
          w_res, b_res_r, w_side, b_side_r, cb_ref, csq_in,
          out_ref, loss_ref, bd_ref, bg_ref, csq_ref, bcol_ref, *, blk):
    i = pl.program_id(0)

    @pl.when(i == 0)
    def _build():
        bd_ref[...] = jnp.zeros((LEVELS, CHANNEL, KT), jnp.float32)
        bg_ref[...] = jnp.zeros((LEVELS, KT, CHANNEL), jnp.float32)
        for l in range(LEVELS):
            for s in range(SEG):
                cb = cb_ref[l, s]
                bg_ref[l, K * s:K * (s + 1), D * s:D * (s + 1)] = cb
                bd_ref[l, D * s:D * (s + 1), K * s:K * (s + 1)] = \
                    jnp.float32(-2.0) * cb.T
        eye_c = (jax.lax.broadcasted_iota(jnp.int32, (CHANNEL, CHANNEL), 0)
                 == jax.lax.broadcasted_iota(
                     jnp.int32, (CHANNEL, CHANNEL), 1)).astype(jnp.float32)
        eye_k = (jax.lax.broadcasted_iota(jnp.int32, (K, K), 0)
                 == jax.lax.broadcasted_iota(
                     jnp.int32, (K, K), 1)).astype(jnp.float32)
        for j, br in enumerate((b_enc_r, b_q_r, b_lat_r, b_deq_r,
                                b_res_r, b_side_r)):
            for l in range(LEVELS):
                bcol_ref[j, l] = _mm(eye_c, br[l:l + 1, :], ((1,), (1,)))
        for l in range(LEVELS):
            for s in range(SEG):
                csq_ref[l, K * s:K * (s + 1)] = \
                    _mm(eye_k, csq_in[l, s:s + 1, :], ((1,), (1,)))

    b_enc, b_q, b_lat, b_deq, b_res, b_side = (
        bcol_ref[0], bcol_ref[1], bcol_ref[2],
        bcol_ref[3], bcol_ref[4], bcol_ref[5])

    cur = x_ref[...]
    iota_f = jax.lax.broadcasted_iota(
        jnp.int32, (K, blk), 0).astype(jnp.float32)

    qs_, crosses = [], []
    for l in range(LEVELS):
        z = _mm(w_enc[l], cur, ((1,), (0,))) + b_enc[l]
        q = _mm(w_q[l], z, ((1,), (0,))) + b_q[l]
        crosses.append(_mm(bd_ref[l], q, ((0,), (0,))))
        qs_.append(q)
        if l < LEVELS - 1:
            cur = _mm(w_lat[l], z, ((1,), (0,))) + b_lat[l]

    pipes = [(l, s) for l in range(LEVELS) for s in range(SEG)]
    ds_all, mn_all, masked_all, idx_all, oh_all = {}, {}, {}, {}, {}
    for l, s in pipes:
        ds_all[l, s] = crosses[l][K * s:K * (s + 1), :] \
            + csq_ref[l][K * s:K * (s + 1), :]
    for l, s in pipes:
        mn_all[l, s] = jnp.min(ds_all[l, s], axis=0, keepdims=True)
    for l, s in pipes:
        masked_all[l, s] = jnp.where(
            ds_all[l, s] == mn_all[l, s], iota_f, jnp.float32(K))
    for l, s in pipes:
        idx_all[l, s] = jnp.min(masked_all[l, s], axis=0, keepdims=True)
    for l, s in pipes:
        oh_all[l, s] = (iota_f == idx_all[l, s]).astype(jnp.float32)

    loss = jnp.float32(0.0)
    hards = []
    for l in range(LEVELS):
        oh = jnp.concatenate([oh_all[l, s] for s in range(SEG)], axis=0)
        hard = _mm(bg_ref[l], oh, ((0,), (0,)))
        df = qs_[l] - hard
        loss = loss + jnp.sum(df * df)
        hards.append(hard)

    t = _mm(w_deq[2], hards[2], ((1,), (0,))) + b_deq[2] + b_side[2]
    y = _mm(w_res[2], t, ((1,), (0,))) + b_res[2]
    for l in (1, 0):
        t = (_mm(w_deq[l], hards[l], ((1,), (0,))) + b_deq[l]
             + _mm(w_side[l], y, ((1,), (0,))) + b_side[l])
        y = _mm(w_res[l], t, ((1,), (0,))) + b_res[l]
    out_ref[...] = y.T

    @pl.when(i == 0)
    def _init():
        loss_ref[0, 0] = loss

    @pl.when(i != 0)
    def _acc():
        loss_ref[0, 0] += loss


def kernel(heter_feature_2d, W_enc, b_enc, W_q, b_q, W_lat, b_lat,
           W_deq, b_deq, W_res, b_res, W_side, b_side, codebooks):
    Bq, C, Hq, Wq_ = heter_feature_2d.shape
    n = Bq * Hq * Wq_
    x = heter_feature_2d.reshape(C, n)
    blk = 4096
    grid = n // blk

    csq = jnp.sum(codebooks * codebooks, axis=-1)

    full = lambda shape: pl.BlockSpec(shape, lambda i: (0,) * len(shape))
    out, loss = pl.pallas_call(
        functools.partial(_body, blk=blk),
        grid=(grid,),
        in_specs=[
            pl.BlockSpec((C, blk), lambda i: (0, i)),
            full((LEVELS, C, C)), full((LEVELS, C)),
            full((LEVELS, C, C)), full((LEVELS, C)),
            full((LEVELS, C, C)), full((LEVELS, C)),
            full((LEVELS, C, C)), full((LEVELS, C)),
            full((LEVELS, C, C)), full((LEVELS, C)),
            full((LEVELS, C, C)), full((LEVELS, C)),
            full((LEVELS, SEG, K, D)), full((LEVELS, SEG, K)),
        ],
        out_specs=[
            pl.BlockSpec((blk, C), lambda i: (i, 0)),
            pl.BlockSpec((1, 1), lambda i: (0, 0),
                         memory_space=pltpu.SMEM),
        ],
        out_shape=[
            jax.ShapeDtypeStruct((n, C), jnp.float32),
            jax.ShapeDtypeStruct((1, 1), jnp.float32),
        ],
        scratch_shapes=[
            pltpu.VMEM((LEVELS, C, KT), jnp.float32),
            pltpu.VMEM((LEVELS, KT, C), jnp.float32),
            pltpu.VMEM((LEVELS, KT, 1), jnp.float32),
            pltpu.VMEM((6, LEVELS, C, 1), jnp.float32),
        ],
    )(x, W_enc, b_enc, W_q, b_q, W_lat, b_lat,
      W_deq, b_deq, W_res, b_res, W_side, b_side,
      codebooks, csq)

    restored = out.reshape(Bq, Hq, Wq_, C)
    codebook_loss = loss[0, 0] * jnp.float32(1.25) / jnp.float32(n * C)
    return (restored, codebook_loss)
